# Initial kernel scaffold; baseline (speedup 1.0000x reference)
import jax
import jax.numpy as jnp
from jax.experimental import pallas as pl


def kernel(f, indices, C_L, Z_init_II, tok_idx, W_l, W_m, rms_w, W_z, W1, W2, W3):
    raise NotImplementedError("write your pallas kernel here")


# speedup vs baseline: 12.8245x; 12.8245x over previous
import dataclasses
import functools

import jax
import jax.numpy as jnp
from jax import lax
from jax.experimental import pallas as pl
from jax.experimental.pallas import tpu as pltpu
from jax.experimental.pallas import tpu_sc as plsc

_EPS = 1.1920928955078125e-07

_NC, _NS = 2, 16
_NW = _NC * _NS
_CW = 128


def _proj_body(c_ref, w_ref, sl_ref, sm_ref):
    y = jnp.maximum(c_ref[...], 0.0)
    s = jnp.dot(y, w_ref[...], preferred_element_type=jnp.float32)
    ca = sl_ref.shape[1]
    sl_ref[...] = s[:, :ca]
    sm_ref[...] = s[:, ca:]


def _zp_body(z_ref, g_ref, w_ref, o_ref):
    x = z_ref[...]
    ms = jnp.mean(x * x, axis=1, keepdims=True)
    y = x * lax.rsqrt(ms + _EPS) * g_ref[...]
    o_ref[...] = jnp.dot(y, w_ref[...], preferred_element_type=jnp.float32)


def _mlp_body(smg_ref, zpg_ref, sl_ref, b1_ref, b2_ref, b3_ref, o_ref):
    sl = sl_ref[...]
    slt = jnp.concatenate([sl] * 32, axis=1)
    p = smg_ref[...] + zpg_ref[...] + slt
    b1, b2, b3 = b1_ref[...], b2_ref[...], b3_ref[...]
    for g in range(p.shape[1] // 128):
        pg = p[:, g * 128:(g + 1) * 128]
        h = jnp.dot(jnp.maximum(pg, 0.0), b1, preferred_element_type=jnp.float32)
        h = jnp.dot(jnp.maximum(h, 0.0), b2, preferred_element_type=jnp.float32)
        h = jnp.dot(jnp.maximum(h, 0.0), b3, preferred_element_type=jnp.float32)
        o_ref[:, g * 128:(g + 1) * 128] = pg + h


def _sc_gather(ind, tok, sm_all, zp):
    n = ind.shape[0]
    l_tot = tok.shape[0]
    i_tot = 512
    ca = sm_all.shape[1]
    per_w = n // _NW
    nch = per_w // _CW
    mesh = plsc.VectorSubcoreMesh(core_axis_name="c", subcore_axis_name="s",
                                  num_cores=_NC, num_subcores=_NS)
    cp = pltpu.CompilerParams()
    if "needs_layout_passes" in pltpu.CompilerParams.__dataclass_fields__:
        cp = dataclasses.replace(cp, needs_layout_passes=False)
    if "use_tc_tiling_on_sc" in pltpu.CompilerParams.__dataclass_fields__:
        cp = dataclasses.replace(cp, use_tc_tiling_on_sc=False)

    @functools.partial(
        pl.kernel,
        compiler_params=cp,
        out_type=(jax.ShapeDtypeStruct((n, ca), jnp.float32),
                  jax.ShapeDtypeStruct((n, ca), jnp.float32)),
        mesh=mesh,
        scratch_types=[
            pltpu.VMEM((l_tot,), jnp.int32),
            pltpu.VMEM((_CW,), jnp.int32),
            pltpu.VMEM((_CW,), jnp.int32),
            pltpu.VMEM((_CW, ca), jnp.float32),
            pltpu.VMEM((_CW, ca), jnp.float32),
            pltpu.SemaphoreType.DMA,
            pltpu.SemaphoreType.DMA,
        ],
    )
    def sck(ind_hbm, tok_hbm, sm_hbm, zp_hbm, osm_hbm, ozp_hbm,
            tok_v, idx_v, flat_v, a_v, b_v, sem_a, sem_b):
        wid = lax.axis_index("s") * _NC + lax.axis_index("c")
        pltpu.sync_copy(tok_hbm, tok_v)

        @pl.loop(0, nch)
        def _chunk(ci):
            base = wid * per_w + ci * _CW
            pltpu.sync_copy(ind_hbm.at[pl.ds(base, _CW)], idx_v)

            @pl.loop(0, _CW, step=16)
            def _vec(i):
                jv = idx_v[pl.ds(i, 16)]
                jv = jnp.minimum(jnp.maximum(jv, 0), l_tot - 1)
                tv = plsc.load_gather(tok_v, [jv])
                pos = base + i + lax.iota(jnp.int32, 16)
                lv = lax.shift_right_logical(pos, 5)
                qv = plsc.load_gather(tok_v, [lv])
                flat_v[pl.ds(i, 16)] = qv * i_tot + tv
                idx_v[pl.ds(i, 16)] = jv

            cp_a = pltpu.async_copy(sm_hbm.at[idx_v], a_v, sem_a)
            cp_b = pltpu.async_copy(zp_hbm.at[flat_v], b_v, sem_b)
            cp_a.wait()
            cp_b.wait()
            pltpu.sync_copy(a_v, osm_hbm.at[pl.ds(base, _CW)])
            pltpu.sync_copy(b_v, ozp_hbm.at[pl.ds(base, _CW)])

    return sck(ind, tok, sm_all, zp)


def kernel(f, indices, C_L, Z_init_II, tok_idx, W_l, W_m, rms_w, W_z, W1, W2, W3):
    d, l, k = indices.shape
    i_tot = Z_init_II.shape[0]
    ct = C_L.shape[-1]
    ca = W_l.shape[0]
    n = d * l * k

    c2 = C_L.reshape(l, ct)
    zf = Z_init_II.reshape(i_tot * i_tot, ct)
    ind = indices.reshape(n)
    wcat = jnp.concatenate([W_l.T, W_m.T], axis=1)

    sl, sm_all = pl.pallas_call(
        _proj_body,
        out_shape=(jax.ShapeDtypeStruct((l, ca), jnp.float32),
                   jax.ShapeDtypeStruct((l, ca), jnp.float32)),
    )(c2, wcat)

    zrows = 4096
    zp = pl.pallas_call(
        _zp_body,
        grid=(i_tot * i_tot // zrows,),
        in_specs=[
            pl.BlockSpec((zrows, ct), lambda i: (i, 0)),
            pl.BlockSpec((1, ct), lambda i: (0, 0)),
            pl.BlockSpec((ct, ca), lambda i: (0, 0)),
        ],
        out_specs=pl.BlockSpec((zrows, ca), lambda i: (i, 0)),
        out_shape=jax.ShapeDtypeStruct((i_tot * i_tot, ca), jnp.float32),
    )(zf, rms_w.reshape(1, ct), W_z.T)

    smg, zpg = _sc_gather(ind, tok_idx, sm_all, zp)

    eye8 = jnp.eye(8, dtype=jnp.float32)
    b1 = jnp.kron(eye8, W1.T)
    b2 = jnp.kron(eye8, W2.T)
    b3 = jnp.kron(eye8, W3.T)

    rows = 512
    wide = k * ca
    out = pl.pallas_call(
        _mlp_body,
        grid=(l // rows,),
        in_specs=[
            pl.BlockSpec((rows, wide), lambda i: (i, 0)),
            pl.BlockSpec((rows, wide), lambda i: (i, 0)),
            pl.BlockSpec((rows, ca), lambda i: (i, 0)),
            pl.BlockSpec((128, 128), lambda i: (0, 0)),
            pl.BlockSpec((128, 128), lambda i: (0, 0)),
            pl.BlockSpec((128, 128), lambda i: (0, 0)),
        ],
        out_specs=pl.BlockSpec((rows, wide), lambda i: (i, 0)),
        out_shape=jax.ShapeDtypeStruct((l, wide), jnp.float32),
    )(smg.reshape(l, wide), zpg.reshape(l, wide), sl, b1, b2, b3)

    return out.reshape(d, l, k, ca)
